# baseline (speedup 1.0000x reference)
import functools

import jax
import jax.numpy as jnp
import numpy as np
from jax import lax
from jax.experimental import pallas as pl
from jax.experimental.pallas import tpu as pltpu
from jax.experimental.pallas import tpu_sc as plsc

V2, K, D = 100000, 1000, 64
RB = 2000
NBLK = V2 // RB
NBUF = 4

BATCH, SEQ = 4096, 50
B = BATCH * SEQ
NC, NS = 2, 16
NW = NC * NS
BPW = B // NW
CHUNK = 128
NCHUNK = BPW // CHUNK

_PERM = np.empty((D,), dtype=np.int32)
for _g in range(D // 32):
    for _i in range(16):
        _PERM[32 * _g + 2 * _i] = 32 * _g + _i
        _PERM[32 * _g + 2 * _i + 1] = 32 * _g + 16 + _i


def _table_body(m_hbm, l1_ref, out_ref, bufs, sems):
    i = pl.program_id(0)

    def start(blk, slot):
        pltpu.make_async_copy(
            m_hbm.at[pl.ds(blk * RB, RB), :], bufs.at[slot], sems.at[slot]
        ).start()

    @pl.when(i == 0)
    def _():
        for b in range(NBUF):
            start(b, b)

    @pl.when((i > 0) & (i + NBUF - 1 < NBLK))
    def _():
        start(i + NBUF - 1, (i + NBUF - 1) % NBUF)

    l1 = l1_ref[...]
    for b in range(NBUF):
        @pl.when(i % NBUF == b)
        def _(b=b):
            pltpu.make_async_copy(
                m_hbm.at[pl.ds(0, RB), :], bufs.at[b], sems.at[b]
            ).wait()
            m = bufs[b]
            mx = jnp.max(m, axis=1, keepdims=True)
            e = jnp.exp(m - mx)
            s = jnp.sum(e, axis=1, keepdims=True)
            acc = jnp.dot(e, l1, preferred_element_type=jnp.float32)
            out_ref[...] = (acc / s).astype(jnp.bfloat16)


def _build_table(map_weights, l1_perm):
    return pl.pallas_call(
        _table_body,
        grid=(NBLK,),
        in_specs=[
            pl.BlockSpec(memory_space=pl.ANY),
            pl.BlockSpec((K, D), lambda i: (0, 0)),
        ],
        out_specs=pl.BlockSpec((RB, D), lambda i: (i, 0)),
        out_shape=jax.ShapeDtypeStruct((V2, D), jnp.bfloat16),
        scratch_shapes=[
            pltpu.VMEM((NBUF, RB, K), jnp.float32),
            pltpu.SemaphoreType.DMA((NBUF,)),
        ],
    )(map_weights, l1_perm)


def _gather_body(table_hbm, x_hbm, out_hbm, idx_v, g_v, o_v, gsem, wsem):
    wid = lax.axis_index("s") * NC + lax.axis_index("c")
    pltpu.sync_copy(x_hbm.at[wid], idx_v)
    base = wid * BPW

    def g_copy(j, slot):
        return pltpu.make_async_copy(
            table_hbm.at[idx_v.at[j]], g_v.at[slot], gsem.at[slot])

    def w_copy(j, slot):
        return pltpu.make_async_copy(
            o_v.at[slot], out_hbm.at[pl.ds(base + j * CHUNK, CHUNK)],
            wsem.at[slot])

    def convert(slot):
        def row(r, c):
            for g in range(D // 32):
                v = g_v[slot, r, pl.ds(32 * g, 32)]
                a, b = plsc.unpack(
                    v, format=plsc.PackFormat.INTERLEAVED,
                    preferred_element_type=jnp.float32)
                o_v[slot, r, pl.ds(32 * g, 16)] = a
                o_v[slot, r, pl.ds(32 * g + 16, 16)] = b
            return c
        lax.fori_loop(0, CHUNK, row, 0, unroll=8)

    g_copy(0, 0).start()

    def body(j, carry):
        slot = j % 2
        nslot = (j + 1) % 2

        @pl.when(j + 1 < NCHUNK)
        def _():
            g_copy(j + 1, nslot).start()

        g_copy(j, slot).wait()

        @pl.when(j >= 2)
        def _():
            w_copy(j - 2, slot).wait()

        convert(slot)
        w_copy(j, slot).start()
        return carry

    lax.fori_loop(0, NCHUNK, body, 0)
    w_copy(NCHUNK - 2, NCHUNK % 2).wait()
    w_copy(NCHUNK - 1, (NCHUNK - 1) % 2).wait()


_gather = functools.partial(
    pl.kernel,
    mesh=plsc.VectorSubcoreMesh(core_axis_name="c", subcore_axis_name="s"),
    out_type=jax.ShapeDtypeStruct((B, D), jnp.float32),
    scratch_types=[
        pltpu.VMEM((NCHUNK, CHUNK), jnp.int32),
        pltpu.VMEM((2, CHUNK, D), jnp.bfloat16),
        pltpu.VMEM((2, CHUNK, D), jnp.float32),
        pltpu.SemaphoreType.DMA((2,)),
        pltpu.SemaphoreType.DMA((2,)),
    ],
    compiler_params=pltpu.CompilerParams(use_tc_tiling_on_sc=False, needs_layout_passes=False),
)(_gather_body)


def kernel(x, l1_weights, map_weights):
    l1_perm = l1_weights[:, _PERM]
    table = _build_table(map_weights, l1_perm)
    idx = x.reshape(NW, NCHUNK, CHUNK).astype(jnp.int32)
    out = _gather(table, idx)
    return out.reshape(x.shape[0], x.shape[1], D)


# speedup vs baseline: 1.0566x; 1.0566x over previous
import functools
import jax
import jax.numpy as jnp
from jax import lax
from jax.experimental import pallas as pl
from jax.experimental.pallas import tpu as pltpu
from jax.experimental.pallas import tpu_sc as plsc

V2, K, D = 100000, 1000, 64
RB = 2000
NBLK = V2 // RB
NBUF = 4

BATCH, SEQ = 4096, 50
B = BATCH * SEQ
NC, NS = 2, 16
NW = NC * NS
BPW = B // NW
CHUNK = 128
NCHUNK = BPW // CHUNK


def _table_body(m_hbm, l1_ref, out_ref, bufs, sems):
    i = pl.program_id(0)

    def start(blk, slot):
        pltpu.make_async_copy(
            m_hbm.at[pl.ds(blk * RB, RB), :], bufs.at[slot], sems.at[slot]
        ).start()

    @pl.when(i == 0)
    def _():
        for b in range(NBUF):
            start(b, b)

    @pl.when((i > 0) & (i + NBUF - 1 < NBLK))
    def _():
        start(i + NBUF - 1, (i + NBUF - 1) % NBUF)

    l1 = l1_ref[...]
    for b in range(NBUF):
        @pl.when(i % NBUF == b)
        def _(b=b):
            pltpu.make_async_copy(
                m_hbm.at[pl.ds(0, RB), :], bufs.at[b], sems.at[b]
            ).wait()
            m = bufs[b]
            mx = jnp.max(m, axis=1, keepdims=True)
            e = jnp.exp(m - mx)
            s = jnp.sum(e, axis=1, keepdims=True)
            out_ref[...] = jnp.dot(e, l1, preferred_element_type=jnp.float32) / s


def _build_table(map_weights, l1_weights):
    return pl.pallas_call(
        _table_body,
        grid=(NBLK,),
        in_specs=[
            pl.BlockSpec(memory_space=pl.ANY),
            pl.BlockSpec((K, D), lambda i: (0, 0)),
        ],
        out_specs=pl.BlockSpec((RB, D), lambda i: (i, 0)),
        out_shape=jax.ShapeDtypeStruct((V2, D), jnp.float32),
        scratch_shapes=[
            pltpu.VMEM((NBUF, RB, K), jnp.float32),
            pltpu.SemaphoreType.DMA((NBUF,)),
        ],
    )(map_weights, l1_weights)


def _gather_body(table_hbm, x_hbm, out_hbm, idx_v, rows_v, gsem, wsem):
    wid = lax.axis_index("s") * NC + lax.axis_index("c")
    pltpu.sync_copy(x_hbm.at[wid], idx_v)
    base = wid * BPW

    def g_copy(j, slot):
        return pltpu.make_async_copy(
            table_hbm.at[idx_v.at[j]], rows_v.at[slot], gsem.at[slot])

    def w_copy(j, slot):
        return pltpu.make_async_copy(
            rows_v.at[slot], out_hbm.at[pl.ds(base + j * CHUNK, CHUNK)],
            wsem.at[slot])

    for p in range(3):
        g_copy(p, p).start()

    def body(j, carry):
        slot = j % 4
        nslot = (j + 3) % 4

        @pl.when(j + 3 < NCHUNK)
        def _():
            @pl.when(j >= 1)
            def _():
                w_copy(j - 1, nslot).wait()
            g_copy(j + 3, nslot).start()

        g_copy(j, slot).wait()
        w_copy(j, slot).start()
        return carry

    lax.fori_loop(0, NCHUNK, body, 0)
    for p in range(4):
        w_copy(NCHUNK - 4 + p, (NCHUNK - 4 + p) % 4).wait()


_gather = functools.partial(
    pl.kernel,
    mesh=plsc.VectorSubcoreMesh(core_axis_name="c", subcore_axis_name="s"),
    out_type=jax.ShapeDtypeStruct((B, D), jnp.float32),
    scratch_types=[
        pltpu.VMEM((NCHUNK, CHUNK), jnp.int32),
        pltpu.VMEM((4, CHUNK, D), jnp.float32),
        pltpu.SemaphoreType.DMA((4,)),
        pltpu.SemaphoreType.DMA((4,)),
    ],
    compiler_params=pltpu.CompilerParams(use_tc_tiling_on_sc=False),
)(_gather_body)


def kernel(x, l1_weights, map_weights):
    table = _build_table(map_weights, l1_weights)
    idx = x.reshape(NW, NCHUNK, CHUNK).astype(jnp.int32)
    out = _gather(table, idx)
    return out.reshape(x.shape[0], x.shape[1], D)
